# Initial kernel scaffold; baseline (speedup 1.0000x reference)
import jax
import jax.numpy as jnp
from jax.experimental import pallas as pl


def kernel(x, edge_index, W1, b1, W2, b2, Wc0, bc0, Wc1, bc1, Wp, bp):
    raise NotImplementedError("write your pallas kernel here")


# speedup vs baseline: 55.6949x; 55.6949x over previous
import functools

import jax
import jax.numpy as jnp
from jax import lax
from jax.experimental import pallas as pl
from jax.experimental.pallas import tpu as pltpu
from jax.experimental.pallas import tpu_sc as plsc

N = 100000
E = 3200000
D_IN = 128
H = 16

NC = 2
NS = 16
NW = NC * NS

B = 1024
NSTEP = -(-E // (NW * B))
PER_W = NSTEP * B
EPAD = PER_W * NW
PADROWS = 128
N16 = 100096
SLC = N16 // NS
NPAD = N16 + PADROWS

RB = 2000
GRID = N // RB

_MESH = dict(core_axis_name="c", subcore_axis_name="s", num_cores=NC,
             num_subcores=NS)


def _leaky(v):
    return jnp.where(v > 0, v, 0.01 * v)


def _mlp_body(x_ref, w1_ref, b1_ref, w2_ref, b2_ref, wc_ref, out_ref):
    h1 = _leaky(jnp.dot(x_ref[...], w1_ref[...],
                        preferred_element_type=jnp.float32) + b1_ref[...])
    h2 = _leaky(jnp.dot(h1, w2_ref[...],
                        preferred_element_type=jnp.float32) + b2_ref[...])
    out_ref[...] = jnp.dot(h2, wc_ref[...],
                           preferred_element_type=jnp.float32)


def _tc_mlp(x, W1, b1, W2, b2, Wc0):
    return pl.pallas_call(
        _mlp_body,
        grid=(GRID,),
        in_specs=[
            pl.BlockSpec((RB, D_IN), lambda i: (i, 0)),
            pl.BlockSpec((D_IN, 256), lambda i: (0, 0)),
            pl.BlockSpec((1, 256), lambda i: (0, 0)),
            pl.BlockSpec((256, H), lambda i: (0, 0)),
            pl.BlockSpec((1, H), lambda i: (0, 0)),
            pl.BlockSpec((H, H), lambda i: (0, 0)),
        ],
        out_specs=pl.BlockSpec((RB, H), lambda i: (i, 0)),
        out_shape=jax.ShapeDtypeStruct((N, H), jnp.float32),
    )(x, W1, b1, W2, b2, Wc0)


def _prep_body(degp_ref, hw0_ref, dinv_ref, g0_ref):
    deg = degp_ref[0, :, 0:1] + degp_ref[1, :, 0:1] + 1.0
    dinv = lax.rsqrt(deg)
    dinv_ref[...] = dinv
    g0_ref[...] = hw0_ref[...] * dinv


def _tc_prep(degp, hw0):
    return pl.pallas_call(
        _prep_body,
        grid=(GRID,),
        in_specs=[
            pl.BlockSpec((NC, RB, H), lambda i: (0, i, 0)),
            pl.BlockSpec((RB, H), lambda i: (i, 0)),
        ],
        out_specs=[
            pl.BlockSpec((RB, 1), lambda i: (i, 0)),
            pl.BlockSpec((RB, H), lambda i: (i, 0)),
        ],
        out_shape=[
            jax.ShapeDtypeStruct((N, 1), jnp.float32),
            jax.ShapeDtypeStruct((N, H), jnp.float32),
        ],
    )(degp, hw0)


def _epi_body(acc_ref, g_ref, dinv_ref, b_ref, wc_ref, gn_ref):
    s = acc_ref[0] + acc_ref[1] + g_ref[...]
    h = _leaky(dinv_ref[...] * s + b_ref[...])
    gn_ref[...] = jnp.dot(h, wc_ref[...],
                          preferred_element_type=jnp.float32) * dinv_ref[...]


def _tc_epi(acc, g, dinv, b, Wc_next):
    return pl.pallas_call(
        _epi_body,
        grid=(GRID,),
        in_specs=[
            pl.BlockSpec((NC, RB, H), lambda i: (0, i, 0)),
            pl.BlockSpec((RB, H), lambda i: (i, 0)),
            pl.BlockSpec((RB, 1), lambda i: (i, 0)),
            pl.BlockSpec((1, H), lambda i: (0, 0)),
            pl.BlockSpec((H, H), lambda i: (0, 0)),
        ],
        out_specs=pl.BlockSpec((RB, H), lambda i: (i, 0)),
        out_shape=jax.ShapeDtypeStruct((N, H), jnp.float32),
    )(acc, g, dinv, b, Wc_next)


def _final_body(acc_ref, g_ref, dinv_ref, b_ref, wp_ref, bp_ref,
                out_ref, h_ref):
    s = acc_ref[0] + acc_ref[1] + g_ref[...]
    h = _leaky(dinv_ref[...] * s + b_ref[...])
    h_ref[...] = h
    o = jnp.dot(h, wp_ref[...], preferred_element_type=jnp.float32) + bp_ref[...]
    out_ref[...] = jnp.sum(o, axis=-1, keepdims=True)


def _tc_final(acc, g, dinv, b, Wp, bp):
    return pl.pallas_call(
        _final_body,
        grid=(GRID,),
        in_specs=[
            pl.BlockSpec((NC, RB, H), lambda i: (0, i, 0)),
            pl.BlockSpec((RB, H), lambda i: (i, 0)),
            pl.BlockSpec((RB, 1), lambda i: (i, 0)),
            pl.BlockSpec((1, H), lambda i: (0, 0)),
            pl.BlockSpec((H, 2), lambda i: (0, 0)),
            pl.BlockSpec((1, 2), lambda i: (0, 0)),
        ],
        out_specs=[
            pl.BlockSpec((RB, 1), lambda i: (i, 0)),
            pl.BlockSpec((RB, H), lambda i: (i, 0)),
        ],
        out_shape=[
            jax.ShapeDtypeStruct((N, 1), jnp.float32),
            jax.ShapeDtypeStruct((N, H), jnp.float32),
        ],
    )(acc, g, dinv, b, Wp, bp)


def _deg_body(dst_hbm, zeros_hbm, ones_hbm, out_hbm, idx_v, ones_v, acc_sh,
              sem):
    c = lax.axis_index("c")
    s = lax.axis_index("s")
    pltpu.sync_copy(zeros_hbm.at[pl.ds(s * SLC, SLC)],
                    acc_sh.at[pl.ds(s * SLC, SLC)])
    pltpu.sync_copy(ones_hbm, ones_v)
    plsc.subcore_barrier()
    base = (c * NS + s) * PER_W

    def body(i, carry):
        off = base + i * B
        pltpu.sync_copy(dst_hbm.at[pl.ds(off, B)], idx_v)
        pltpu.sync_copy(ones_v, acc_sh.at[idx_v], add=True)
        return carry

    lax.fori_loop(0, NSTEP, body, 0)
    plsc.subcore_barrier()
    pltpu.sync_copy(acc_sh.at[pl.ds(s * SLC, SLC)],
                    out_hbm.at[c, pl.ds(s * SLC, SLC)])


_sc_degree = pl.kernel(
    _deg_body,
    out_type=jax.ShapeDtypeStruct((NC, N16, H), jnp.float32),
    mesh=plsc.VectorSubcoreMesh(**_MESH),
    compiler_params=pltpu.CompilerParams(use_tc_tiling_on_sc=False),
    scratch_types=[
        pltpu.VMEM((B,), jnp.int32),
        pltpu.VMEM((B, H), jnp.float32),
        pltpu.VMEM_SHARED((NPAD, H), jnp.float32),
        pltpu.SemaphoreType.DMA,
    ],
)


def _conv_body(src_hbm, dst_hbm, g_hbm, zeros_hbm, out_hbm,
               isrc_v, idst_v, rows_v, acc_sh, sem):
    c = lax.axis_index("c")
    s = lax.axis_index("s")
    pltpu.sync_copy(zeros_hbm.at[pl.ds(s * SLC, SLC)],
                    acc_sh.at[pl.ds(s * SLC, SLC)])
    plsc.subcore_barrier()
    base = (c * NS + s) * PER_W

    def body(i, carry):
        off = base + i * B
        pltpu.sync_copy(src_hbm.at[pl.ds(off, B)], isrc_v)
        pltpu.sync_copy(dst_hbm.at[pl.ds(off, B)], idst_v)
        pltpu.async_copy(g_hbm.at[isrc_v], rows_v, sem).wait()
        pltpu.sync_copy(rows_v, acc_sh.at[idst_v], add=True)
        return carry

    lax.fori_loop(0, NSTEP, body, 0)
    plsc.subcore_barrier()
    pltpu.sync_copy(acc_sh.at[pl.ds(s * SLC, SLC)],
                    out_hbm.at[c, pl.ds(s * SLC, SLC)])


_sc_conv = pl.kernel(
    _conv_body,
    out_type=jax.ShapeDtypeStruct((NC, N16, H), jnp.float32),
    mesh=plsc.VectorSubcoreMesh(**_MESH),
    compiler_params=pltpu.CompilerParams(use_tc_tiling_on_sc=False),
    scratch_types=[
        pltpu.VMEM((B,), jnp.int32),
        pltpu.VMEM((B,), jnp.int32),
        pltpu.VMEM((B, H), jnp.float32),
        pltpu.VMEM_SHARED((NPAD, H), jnp.float32),
        pltpu.SemaphoreType.DMA,
    ],
)


def kernel(x, edge_index, W1, b1, W2, b2, Wc0, bc0, Wc1, bc1, Wp, bp):
    src = edge_index[0]
    dst = edge_index[1]
    pad = EPAD - E
    pidx = jnp.arange(pad, dtype=jnp.int32)
    src_p = jnp.concatenate([src, pidx % PADROWS])
    dst_p = jnp.concatenate([dst, N16 + (pidx % PADROWS)])

    zeros16 = jnp.zeros((N16, H), jnp.float32)
    ones_b = jnp.ones((B, H), jnp.float32)

    b1r = b1.reshape(1, 256)
    b2r = b2.reshape(1, H)
    bc0r = bc0.reshape(1, H)
    bc1r = bc1.reshape(1, H)
    bpr = bp.reshape(1, 2)

    degp = _sc_degree(dst_p, zeros16, ones_b)
    hw0 = _tc_mlp(x, W1, b1r, W2, b2r, Wc0)
    dinv, g0 = _tc_prep(degp, hw0)
    acc0 = _sc_conv(src_p, dst_p, g0, zeros16)
    g1 = _tc_epi(acc0, g0, dinv, bc0r, Wc1)
    acc1 = _sc_conv(src_p, dst_p, g1, zeros16)
    out2, h = _tc_final(acc1, g1, dinv, bc1r, Wp, bpr)
    return (out2.reshape(N), h)


# speedup vs baseline: 71.6027x; 1.2856x over previous
import jax
import jax.numpy as jnp
from jax import lax
from jax.experimental import pallas as pl
from jax.experimental.pallas import tpu as pltpu
from jax.experimental.pallas import tpu_sc as plsc

N = 100000
E = 3200000
D_IN = 128
H = 16

NC = 2
NS = 16
NW = NC * NS

B = 800
NSTEP = E // (NW * B)
PER_W = NSTEP * B

N16 = 100096
SLC = N16 // NS

RB = 2000
GRID = N // RB

_MESH = dict(core_axis_name="c", subcore_axis_name="s", num_cores=NC,
             num_subcores=NS)
_SC_PARAMS = pltpu.CompilerParams(use_tc_tiling_on_sc=False)


def _leaky(v):
    return jnp.where(v > 0, v, 0.01 * v)


def _mlp_body(x_ref, w1_ref, b1_ref, w2_ref, b2_ref, wc_ref, out_ref):
    h1 = _leaky(jnp.dot(x_ref[...], w1_ref[...],
                        preferred_element_type=jnp.float32) + b1_ref[...])
    h2 = _leaky(jnp.dot(h1, w2_ref[...],
                        preferred_element_type=jnp.float32) + b2_ref[...])
    out_ref[...] = jnp.dot(h2, wc_ref[...],
                           preferred_element_type=jnp.float32)


def _tc_mlp(x, W1, b1, W2, b2, Wc0):
    return pl.pallas_call(
        _mlp_body,
        grid=(GRID,),
        in_specs=[
            pl.BlockSpec((RB, D_IN), lambda i: (i, 0)),
            pl.BlockSpec((D_IN, 256), lambda i: (0, 0)),
            pl.BlockSpec((1, 256), lambda i: (0, 0)),
            pl.BlockSpec((256, H), lambda i: (0, 0)),
            pl.BlockSpec((1, H), lambda i: (0, 0)),
            pl.BlockSpec((H, H), lambda i: (0, 0)),
        ],
        out_specs=pl.BlockSpec((RB, H), lambda i: (i, 0)),
        out_shape=jax.ShapeDtypeStruct((N, H), jnp.float32),
    )(x, W1, b1, W2, b2, Wc0)


def _prep_body(degp_ref, hw0_ref, dinv_ref, g0_ref):
    deg = degp_ref[0, :, 0:1] + degp_ref[1, :, 0:1] + 1.0
    dinv = lax.rsqrt(deg)
    dinv_ref[...] = dinv
    g0_ref[...] = hw0_ref[...] * dinv


def _tc_prep(degp, hw0):
    return pl.pallas_call(
        _prep_body,
        grid=(GRID,),
        in_specs=[
            pl.BlockSpec((NC, RB, H), lambda i: (0, i, 0)),
            pl.BlockSpec((RB, H), lambda i: (i, 0)),
        ],
        out_specs=[
            pl.BlockSpec((RB, 1), lambda i: (i, 0)),
            pl.BlockSpec((RB, H), lambda i: (i, 0)),
        ],
        out_shape=[
            jax.ShapeDtypeStruct((N, 1), jnp.float32),
            jax.ShapeDtypeStruct((N, H), jnp.float32),
        ],
    )(degp, hw0)


def _epi_body(acc_ref, g_ref, dinv_ref, b_ref, wc_ref, gn_ref):
    s = acc_ref[0] + acc_ref[1] + g_ref[...]
    h = _leaky(dinv_ref[...] * s + b_ref[...])
    gn_ref[...] = jnp.dot(h, wc_ref[...],
                          preferred_element_type=jnp.float32) * dinv_ref[...]


def _tc_epi(acc, g, dinv, b, Wc_next):
    return pl.pallas_call(
        _epi_body,
        grid=(GRID,),
        in_specs=[
            pl.BlockSpec((NC, RB, H), lambda i: (0, i, 0)),
            pl.BlockSpec((RB, H), lambda i: (i, 0)),
            pl.BlockSpec((RB, 1), lambda i: (i, 0)),
            pl.BlockSpec((1, H), lambda i: (0, 0)),
            pl.BlockSpec((H, H), lambda i: (0, 0)),
        ],
        out_specs=pl.BlockSpec((RB, H), lambda i: (i, 0)),
        out_shape=jax.ShapeDtypeStruct((N, H), jnp.float32),
    )(acc, g, dinv, b, Wc_next)


def _final_body(acc_ref, g_ref, dinv_ref, b_ref, wp_ref, bp_ref,
                out_ref, h_ref):
    s = acc_ref[0] + acc_ref[1] + g_ref[...]
    h = _leaky(dinv_ref[...] * s + b_ref[...])
    h_ref[...] = h
    o = jnp.dot(h, wp_ref[...], preferred_element_type=jnp.float32) + bp_ref[...]
    out_ref[...] = jnp.sum(o, axis=-1, keepdims=True)


def _tc_final(acc, g, dinv, b, Wp, bp):
    return pl.pallas_call(
        _final_body,
        grid=(GRID,),
        in_specs=[
            pl.BlockSpec((NC, RB, H), lambda i: (0, i, 0)),
            pl.BlockSpec((RB, H), lambda i: (i, 0)),
            pl.BlockSpec((RB, 1), lambda i: (i, 0)),
            pl.BlockSpec((1, H), lambda i: (0, 0)),
            pl.BlockSpec((H, 2), lambda i: (0, 0)),
            pl.BlockSpec((1, 2), lambda i: (0, 0)),
        ],
        out_specs=[
            pl.BlockSpec((RB, 1), lambda i: (i, 0)),
            pl.BlockSpec((RB, H), lambda i: (i, 0)),
        ],
        out_shape=[
            jax.ShapeDtypeStruct((N, 1), jnp.float32),
            jax.ShapeDtypeStruct((N, H), jnp.float32),
        ],
    )(acc, g, dinv, b, Wp, bp)


def _deg_body(ei_hbm, zeros_hbm, ones_hbm, out_hbm,
              idx0, idx1, ones_v, acc_sh, semi0, semi1):
    c = lax.axis_index("c")
    s = lax.axis_index("s")
    pltpu.sync_copy(zeros_hbm.at[pl.ds(s * SLC, SLC)],
                    acc_sh.at[pl.ds(s * SLC, SLC)])
    pltpu.sync_copy(ones_hbm, ones_v)
    plsc.subcore_barrier()
    base = (c * NS + s) * PER_W
    bufs = ((idx0, semi0), (idx1, semi1))

    def start_idx(i, buf):
        idx, semi = buf
        pltpu.async_copy(ei_hbm.at[1, pl.ds(base + i * B, B)], idx, semi)

    def wait_idx(i, buf):
        idx, semi = buf
        pltpu.make_async_copy(ei_hbm.at[1, pl.ds(base + i * B, B)],
                              idx, semi).wait()

    start_idx(0, bufs[0])

    def group(gi, carry):
        for b in range(2):
            i = gi * 2 + b
            nxt = i + 1

            @pl.when(nxt < NSTEP)
            def _():
                start_idx(nxt, bufs[1 - b])

            wait_idx(i, bufs[b])
            pltpu.sync_copy(ones_v, acc_sh.at[bufs[b][0]], add=True)
        return carry

    lax.fori_loop(0, NSTEP // 2, group, 0)
    if NSTEP % 2:
        wait_idx(NSTEP - 1, bufs[(NSTEP - 1) % 2])
        pltpu.sync_copy(ones_v, acc_sh.at[bufs[(NSTEP - 1) % 2][0]], add=True)
    plsc.subcore_barrier()
    pltpu.sync_copy(acc_sh.at[pl.ds(s * SLC, SLC)],
                    out_hbm.at[c, pl.ds(s * SLC, SLC)])


_sc_degree = pl.kernel(
    _deg_body,
    out_type=jax.ShapeDtypeStruct((NC, N16, H), jnp.float32),
    mesh=plsc.VectorSubcoreMesh(**_MESH),
    compiler_params=_SC_PARAMS,
    scratch_types=[
        pltpu.VMEM((B,), jnp.int32),
        pltpu.VMEM((B,), jnp.int32),
        pltpu.VMEM((B, H), jnp.float32),
        pltpu.VMEM_SHARED((N16, H), jnp.float32),
        pltpu.SemaphoreType.DMA,
        pltpu.SemaphoreType.DMA,
    ],
)


def _conv_body(ei_hbm, g_hbm, zeros_hbm, out_hbm,
               isrc0, idst0, rows0, isrc1, idst1, rows1, acc_sh,
               semg0, semg1):
    c = lax.axis_index("c")
    s = lax.axis_index("s")
    pltpu.sync_copy(zeros_hbm.at[pl.ds(s * SLC, SLC)],
                    acc_sh.at[pl.ds(s * SLC, SLC)])
    plsc.subcore_barrier()
    base = (c * NS + s) * PER_W
    bufs = ((isrc0, idst0, rows0, semg0), (isrc1, idst1, rows1, semg1))

    def load_and_gather(i, buf):
        isrc, idst, rows, semg = buf
        off = base + i * B
        pltpu.sync_copy(ei_hbm.at[0, pl.ds(off, B)], isrc)
        pltpu.sync_copy(ei_hbm.at[1, pl.ds(off, B)], idst)
        pltpu.async_copy(g_hbm.at[isrc], rows, semg)

    load_and_gather(0, bufs[0])

    def group(gi, carry):
        for b in range(2):
            i = gi * 2 + b
            nxt = i + 1

            @pl.when(nxt < NSTEP)
            def _():
                load_and_gather(nxt, bufs[1 - b])

            isrc, idst, rows, semg = bufs[b]
            pltpu.make_async_copy(g_hbm.at[isrc], rows, semg).wait()
            pltpu.sync_copy(rows, acc_sh.at[idst], add=True)
        return carry

    lax.fori_loop(0, NSTEP // 2, group, 0)
    if NSTEP % 2:
        isrc, idst, rows, semg = bufs[(NSTEP - 1) % 2]
        pltpu.make_async_copy(g_hbm.at[isrc], rows, semg).wait()
        pltpu.sync_copy(rows, acc_sh.at[idst], add=True)
    plsc.subcore_barrier()
    pltpu.sync_copy(acc_sh.at[pl.ds(s * SLC, SLC)],
                    out_hbm.at[c, pl.ds(s * SLC, SLC)])


_sc_conv = pl.kernel(
    _conv_body,
    out_type=jax.ShapeDtypeStruct((NC, N16, H), jnp.float32),
    mesh=plsc.VectorSubcoreMesh(**_MESH),
    compiler_params=_SC_PARAMS,
    scratch_types=[
        pltpu.VMEM((B,), jnp.int32),
        pltpu.VMEM((B,), jnp.int32),
        pltpu.VMEM((B, H), jnp.float32),
        pltpu.VMEM((B,), jnp.int32),
        pltpu.VMEM((B,), jnp.int32),
        pltpu.VMEM((B, H), jnp.float32),
        pltpu.VMEM_SHARED((N16, H), jnp.float32),
        pltpu.SemaphoreType.DMA,
        pltpu.SemaphoreType.DMA,
    ],
)


def kernel(x, edge_index, W1, b1, W2, b2, Wc0, bc0, Wc1, bc1, Wp, bp):
    zeros16 = jnp.zeros((N16, H), jnp.float32)
    ones_b = jnp.ones((B, H), jnp.float32)

    b1r = b1.reshape(1, 256)
    b2r = b2.reshape(1, H)
    bc0r = bc0.reshape(1, H)
    bc1r = bc1.reshape(1, H)
    bpr = bp.reshape(1, 2)

    degp = _sc_degree(edge_index, zeros16, ones_b)
    hw0 = _tc_mlp(x, W1, b1r, W2, b2r, Wc0)
    dinv, g0 = _tc_prep(degp, hw0)
    acc0 = _sc_conv(edge_index, g0, zeros16)
    g1 = _tc_epi(acc0, g0, dinv, bc0r, Wc1)
    acc1 = _sc_conv(edge_index, g1, zeros16)
    out2, h = _tc_final(acc1, g1, dinv, bc1r, Wp, bpr)
    return (out2.reshape(N), h)


# speedup vs baseline: 76.7151x; 1.0714x over previous
import jax
import jax.numpy as jnp
from jax import lax
from jax.experimental import pallas as pl
from jax.experimental.pallas import tpu as pltpu
from jax.experimental.pallas import tpu_sc as plsc

N = 100000
E = 3200000
D_IN = 128
H = 16

NC = 2
NS = 16
NW = NC * NS

B = 800
NSTEP = E // (NW * B)
PER_W = NSTEP * B

N16 = 100096
SLC = N16 // NS

RB = 2000
GRID = N // RB

_MESH = dict(core_axis_name="c", subcore_axis_name="s", num_cores=NC,
             num_subcores=NS)
_SC_PARAMS = pltpu.CompilerParams(use_tc_tiling_on_sc=False)


def _leaky(v):
    return jnp.where(v > 0, v, 0.01 * v)


def _mlp_body(x_ref, w1_ref, b1_ref, w2_ref, b2_ref, wc_ref, out_ref):
    h1 = _leaky(jnp.dot(x_ref[...], w1_ref[...],
                        preferred_element_type=jnp.float32) + b1_ref[...])
    h2 = _leaky(jnp.dot(h1, w2_ref[...],
                        preferred_element_type=jnp.float32) + b2_ref[...])
    out_ref[...] = jnp.dot(h2, wc_ref[...],
                           preferred_element_type=jnp.float32)


def _tc_mlp(x, W1, b1, W2, b2, Wc0):
    return pl.pallas_call(
        _mlp_body,
        grid=(GRID,),
        in_specs=[
            pl.BlockSpec((RB, D_IN), lambda i: (i, 0)),
            pl.BlockSpec((D_IN, 256), lambda i: (0, 0)),
            pl.BlockSpec((1, 256), lambda i: (0, 0)),
            pl.BlockSpec((256, H), lambda i: (0, 0)),
            pl.BlockSpec((1, H), lambda i: (0, 0)),
            pl.BlockSpec((H, H), lambda i: (0, 0)),
        ],
        out_specs=pl.BlockSpec((RB, H), lambda i: (i, 0)),
        out_shape=jax.ShapeDtypeStruct((N, H), jnp.float32),
    )(x, W1, b1, W2, b2, Wc0)


def _prep_body(degp_ref, hw0_ref, dinv_ref, g0_ref):
    deg = degp_ref[0, :, 0:1] + degp_ref[1, :, 0:1] + 1.0
    dinv = lax.rsqrt(deg)
    dinv_ref[...] = dinv
    g0_ref[...] = hw0_ref[...] * dinv


def _tc_prep(degp, hw0):
    return pl.pallas_call(
        _prep_body,
        grid=(GRID,),
        in_specs=[
            pl.BlockSpec((NC, RB, H), lambda i: (0, i, 0)),
            pl.BlockSpec((RB, H), lambda i: (i, 0)),
        ],
        out_specs=[
            pl.BlockSpec((RB, 1), lambda i: (i, 0)),
            pl.BlockSpec((RB, H), lambda i: (i, 0)),
        ],
        out_shape=[
            jax.ShapeDtypeStruct((N, 1), jnp.float32),
            jax.ShapeDtypeStruct((N, H), jnp.float32),
        ],
    )(degp, hw0)


def _epi_body(acc_ref, g_ref, dinv_ref, b_ref, wc_ref, gn_ref):
    s = acc_ref[0] + acc_ref[1] + g_ref[...]
    h = _leaky(dinv_ref[...] * s + b_ref[...])
    gn_ref[...] = jnp.dot(h, wc_ref[...],
                          preferred_element_type=jnp.float32) * dinv_ref[...]


def _tc_epi(acc, g, dinv, b, Wc_next):
    return pl.pallas_call(
        _epi_body,
        grid=(GRID,),
        in_specs=[
            pl.BlockSpec((NC, RB, H), lambda i: (0, i, 0)),
            pl.BlockSpec((RB, H), lambda i: (i, 0)),
            pl.BlockSpec((RB, 1), lambda i: (i, 0)),
            pl.BlockSpec((1, H), lambda i: (0, 0)),
            pl.BlockSpec((H, H), lambda i: (0, 0)),
        ],
        out_specs=pl.BlockSpec((RB, H), lambda i: (i, 0)),
        out_shape=jax.ShapeDtypeStruct((N, H), jnp.float32),
    )(acc, g, dinv, b, Wc_next)


def _final_body(acc_ref, g_ref, dinv_ref, b_ref, wp_ref, bp_ref,
                out_ref, h_ref):
    s = acc_ref[0] + acc_ref[1] + g_ref[...]
    h = _leaky(dinv_ref[...] * s + b_ref[...])
    h_ref[...] = h
    o = jnp.dot(h, wp_ref[...], preferred_element_type=jnp.float32) + bp_ref[...]
    out_ref[...] = jnp.sum(o, axis=-1, keepdims=True)


def _tc_final(acc, g, dinv, b, Wp, bp):
    return pl.pallas_call(
        _final_body,
        grid=(GRID,),
        in_specs=[
            pl.BlockSpec((NC, RB, H), lambda i: (0, i, 0)),
            pl.BlockSpec((RB, H), lambda i: (i, 0)),
            pl.BlockSpec((RB, 1), lambda i: (i, 0)),
            pl.BlockSpec((1, H), lambda i: (0, 0)),
            pl.BlockSpec((H, 2), lambda i: (0, 0)),
            pl.BlockSpec((1, 2), lambda i: (0, 0)),
        ],
        out_specs=[
            pl.BlockSpec((RB, 1), lambda i: (i, 0)),
            pl.BlockSpec((RB, H), lambda i: (i, 0)),
        ],
        out_shape=[
            jax.ShapeDtypeStruct((N, 1), jnp.float32),
            jax.ShapeDtypeStruct((N, H), jnp.float32),
        ],
    )(acc, g, dinv, b, Wp, bp)


def _deg_body(ei_hbm, zeros_hbm, ones_hbm, out_hbm,
              idx0, idx1, ones_v, acc_sh, semi0, semi1):
    c = lax.axis_index("c")
    s = lax.axis_index("s")
    pltpu.sync_copy(zeros_hbm.at[pl.ds(s * SLC, SLC)],
                    acc_sh.at[pl.ds(s * SLC, SLC)])
    pltpu.sync_copy(ones_hbm, ones_v)
    plsc.subcore_barrier()
    base = (c * NS + s) * PER_W
    bufs = ((idx0, semi0), (idx1, semi1))

    def start_idx(i, buf):
        idx, semi = buf
        pltpu.async_copy(ei_hbm.at[1, pl.ds(base + i * B, B)], idx, semi)

    def wait_idx(i, buf):
        idx, semi = buf
        pltpu.make_async_copy(ei_hbm.at[1, pl.ds(base + i * B, B)],
                              idx, semi).wait()

    def chunk(i, b):
        wait_idx(i, bufs[b])
        pltpu.sync_copy(ones_v, acc_sh.at[bufs[b][0]], add=True)

        @pl.when(i + 2 < NSTEP)
        def _():
            start_idx(i + 2, bufs[b])

    start_idx(0, bufs[0])
    if NSTEP > 1:
        start_idx(1, bufs[1])

    def group(gi, carry):
        for b in range(2):
            chunk(gi * 2 + b, b)
        return carry

    lax.fori_loop(0, NSTEP // 2, group, 0)
    if NSTEP % 2:
        chunk(NSTEP - 1, (NSTEP - 1) % 2)
    plsc.subcore_barrier()
    pltpu.sync_copy(acc_sh.at[pl.ds(s * SLC, SLC)],
                    out_hbm.at[c, pl.ds(s * SLC, SLC)])


_sc_degree = pl.kernel(
    _deg_body,
    out_type=jax.ShapeDtypeStruct((NC, N16, H), jnp.float32),
    mesh=plsc.VectorSubcoreMesh(**_MESH),
    compiler_params=_SC_PARAMS,
    scratch_types=[
        pltpu.VMEM((B,), jnp.int32),
        pltpu.VMEM((B,), jnp.int32),
        pltpu.VMEM((B, H), jnp.float32),
        pltpu.VMEM_SHARED((N16, H), jnp.float32),
        pltpu.SemaphoreType.DMA,
        pltpu.SemaphoreType.DMA,
    ],
)


def _conv_body(ei_hbm, g_hbm, zeros_hbm, out_hbm,
               isrc0, idst0, rows0, isrc1, idst1, rows1, acc_sh,
               semg0, semg1, semi0, semi1):
    c = lax.axis_index("c")
    s = lax.axis_index("s")
    pltpu.sync_copy(zeros_hbm.at[pl.ds(s * SLC, SLC)],
                    acc_sh.at[pl.ds(s * SLC, SLC)])
    plsc.subcore_barrier()
    base = (c * NS + s) * PER_W
    bufs = ((isrc0, idst0, rows0, semg0, semi0),
            (isrc1, idst1, rows1, semg1, semi1))

    def start_idx(i, buf):
        isrc, idst, rows, semg, semi = buf
        off = base + i * B
        pltpu.async_copy(ei_hbm.at[0, pl.ds(off, B)], isrc, semi)
        pltpu.async_copy(ei_hbm.at[1, pl.ds(off, B)], idst, semi)

    def wait_idx(i, buf):
        isrc, idst, rows, semg, semi = buf
        off = base + i * B
        pltpu.make_async_copy(ei_hbm.at[0, pl.ds(off, B)], isrc, semi).wait()
        pltpu.make_async_copy(ei_hbm.at[1, pl.ds(off, B)], idst, semi).wait()

    def chunk(i, b):
        nxt = i + 1

        @pl.when(nxt < NSTEP)
        def _():
            wait_idx(nxt, bufs[1 - b])
            pltpu.async_copy(g_hbm.at[bufs[1 - b][0]], bufs[1 - b][2],
                             bufs[1 - b][3])

        isrc, idst, rows, semg, semi = bufs[b]
        pltpu.make_async_copy(g_hbm.at[isrc], rows, semg).wait()
        pltpu.sync_copy(rows, acc_sh.at[idst], add=True)

        @pl.when(i + 2 < NSTEP)
        def _():
            start_idx(i + 2, bufs[b])

    start_idx(0, bufs[0])
    wait_idx(0, bufs[0])
    pltpu.async_copy(g_hbm.at[bufs[0][0]], bufs[0][2], bufs[0][3])
    if NSTEP > 1:
        start_idx(1, bufs[1])

    def group(gi, carry):
        for b in range(2):
            chunk(gi * 2 + b, b)
        return carry

    lax.fori_loop(0, NSTEP // 2, group, 0)
    if NSTEP % 2:
        chunk(NSTEP - 1, (NSTEP - 1) % 2)
    plsc.subcore_barrier()
    pltpu.sync_copy(acc_sh.at[pl.ds(s * SLC, SLC)],
                    out_hbm.at[c, pl.ds(s * SLC, SLC)])


_sc_conv = pl.kernel(
    _conv_body,
    out_type=jax.ShapeDtypeStruct((NC, N16, H), jnp.float32),
    mesh=plsc.VectorSubcoreMesh(**_MESH),
    compiler_params=_SC_PARAMS,
    scratch_types=[
        pltpu.VMEM((B,), jnp.int32),
        pltpu.VMEM((B,), jnp.int32),
        pltpu.VMEM((B, H), jnp.float32),
        pltpu.VMEM((B,), jnp.int32),
        pltpu.VMEM((B,), jnp.int32),
        pltpu.VMEM((B, H), jnp.float32),
        pltpu.VMEM_SHARED((N16, H), jnp.float32),
        pltpu.SemaphoreType.DMA,
        pltpu.SemaphoreType.DMA,
        pltpu.SemaphoreType.DMA,
        pltpu.SemaphoreType.DMA,
    ],
)


def kernel(x, edge_index, W1, b1, W2, b2, Wc0, bc0, Wc1, bc1, Wp, bp):
    zeros16 = jnp.zeros((N16, H), jnp.float32)
    ones_b = jnp.ones((B, H), jnp.float32)

    b1r = b1.reshape(1, 256)
    b2r = b2.reshape(1, H)
    bc0r = bc0.reshape(1, H)
    bc1r = bc1.reshape(1, H)
    bpr = bp.reshape(1, 2)

    degp = _sc_degree(edge_index, zeros16, ones_b)
    hw0 = _tc_mlp(x, W1, b1r, W2, b2r, Wc0)
    dinv, g0 = _tc_prep(degp, hw0)
    acc0 = _sc_conv(edge_index, g0, zeros16)
    g1 = _tc_epi(acc0, g0, dinv, bc0r, Wc1)
    acc1 = _sc_conv(edge_index, g1, zeros16)
    out2, h = _tc_final(acc1, g1, dinv, bc1r, Wp, bpr)
    return (out2.reshape(N), h)


# speedup vs baseline: 132.3675x; 1.7254x over previous
import jax
import jax.numpy as jnp
from jax import lax
from jax.experimental import pallas as pl
from jax.experimental.pallas import tpu as pltpu
from jax.experimental.pallas import tpu_sc as plsc

N = 100000
E = 3200000
D_IN = 128
H = 16

NC = 2
NS = 16
NW = NC * NS

B = 800
NSTEP = E // (NW * B)
PER_W = NSTEP * B

N16 = 100096
SLC = N16 // NS

M16 = N16 // 8
RBP = M16 // 4
GRIDP = M16 // RBP

RB = 2000
GRID = N // RB

_MESH = dict(core_axis_name="c", subcore_axis_name="s", num_cores=NC,
             num_subcores=NS)
_SC_PARAMS = pltpu.CompilerParams(use_tc_tiling_on_sc=False)


def _leaky(v):
    return jnp.where(v > 0, v, 0.01 * v)


def _mlp_body(x_ref, w1_ref, b1_ref, w2_ref, b2_ref, wc_ref, out_ref):
    h1 = _leaky(jnp.dot(x_ref[...], w1_ref[...],
                        preferred_element_type=jnp.float32) + b1_ref[...])
    h2 = _leaky(jnp.dot(h1, w2_ref[...],
                        preferred_element_type=jnp.float32) + b2_ref[...])
    out_ref[...] = jnp.dot(h2, wc_ref[...],
                           preferred_element_type=jnp.float32)


def _tc_mlp(x, W1, b1, W2, b2, Wc0):
    return pl.pallas_call(
        _mlp_body,
        grid=(GRID,),
        in_specs=[
            pl.BlockSpec((RB, D_IN), lambda i: (i, 0)),
            pl.BlockSpec((D_IN, 256), lambda i: (0, 0)),
            pl.BlockSpec((1, 256), lambda i: (0, 0)),
            pl.BlockSpec((256, H), lambda i: (0, 0)),
            pl.BlockSpec((1, H), lambda i: (0, 0)),
            pl.BlockSpec((H, H), lambda i: (0, 0)),
        ],
        out_specs=pl.BlockSpec((RB, H), lambda i: (i, 0)),
        out_shape=jax.ShapeDtypeStruct((N16, H), jnp.float32),
    )(x, W1, b1, W2, b2, Wc0)


def _prep_body(degp_ref, hw0_ref, dinv_ref, g0_ref):
    deg = degp_ref[0] + degp_ref[1] + 1.0
    dinv = lax.rsqrt(deg)
    dinv_ref[...] = dinv
    g0_ref[...] = hw0_ref[...] * dinv


def _tc_prep(degp, hw0):
    return pl.pallas_call(
        _prep_body,
        grid=(GRIDP,),
        in_specs=[
            pl.BlockSpec((NC, RBP, 128), lambda i: (0, i, 0)),
            pl.BlockSpec((RBP, 128), lambda i: (i, 0)),
        ],
        out_specs=[
            pl.BlockSpec((RBP, 128), lambda i: (i, 0)),
            pl.BlockSpec((RBP, 128), lambda i: (i, 0)),
        ],
        out_shape=[
            jax.ShapeDtypeStruct((M16, 128), jnp.float32),
            jax.ShapeDtypeStruct((M16, 128), jnp.float32),
        ],
    )(degp, hw0)


def _epi_body(acc_ref, g_ref, dinv_ref, b_ref, wcbd_ref, gn_ref):
    s = acc_ref[0] + acc_ref[1] + g_ref[...]
    h = _leaky(dinv_ref[...] * s + b_ref[...])
    gn_ref[...] = jnp.dot(h, wcbd_ref[...],
                          preferred_element_type=jnp.float32) * dinv_ref[...]


def _tc_epi(acc, g, dinv, b8, Wcbd):
    return pl.pallas_call(
        _epi_body,
        grid=(GRIDP,),
        in_specs=[
            pl.BlockSpec((NC, RBP, 128), lambda i: (0, i, 0)),
            pl.BlockSpec((RBP, 128), lambda i: (i, 0)),
            pl.BlockSpec((RBP, 128), lambda i: (i, 0)),
            pl.BlockSpec((1, 128), lambda i: (0, 0)),
            pl.BlockSpec((128, 128), lambda i: (0, 0)),
        ],
        out_specs=pl.BlockSpec((RBP, 128), lambda i: (i, 0)),
        out_shape=jax.ShapeDtypeStruct((M16, 128), jnp.float32),
    )(acc, g, dinv, b8, Wcbd)


def _final_body(acc_ref, g_ref, dinv_ref, b_ref, wpbd_ref, bps_ref,
                out_ref, h_ref):
    s = acc_ref[0] + acc_ref[1] + g_ref[...]
    h = _leaky(dinv_ref[...] * s + b_ref[...])
    h_ref[...] = h
    out_ref[...] = jnp.dot(h, wpbd_ref[...],
                           preferred_element_type=jnp.float32) + bps_ref[...]


def _tc_final(acc, g, dinv, b8, wpbd, bps):
    return pl.pallas_call(
        _final_body,
        grid=(GRIDP,),
        in_specs=[
            pl.BlockSpec((NC, RBP, 128), lambda i: (0, i, 0)),
            pl.BlockSpec((RBP, 128), lambda i: (i, 0)),
            pl.BlockSpec((RBP, 128), lambda i: (i, 0)),
            pl.BlockSpec((1, 128), lambda i: (0, 0)),
            pl.BlockSpec((128, 8), lambda i: (0, 0)),
            pl.BlockSpec((1, 8), lambda i: (0, 0)),
        ],
        out_specs=[
            pl.BlockSpec((RBP, 8), lambda i: (i, 0)),
            pl.BlockSpec((RBP, 128), lambda i: (i, 0)),
        ],
        out_shape=[
            jax.ShapeDtypeStruct((M16, 8), jnp.float32),
            jax.ShapeDtypeStruct((M16, 128), jnp.float32),
        ],
    )(acc, g, dinv, b8, wpbd, bps)


def _deg_body(ei_hbm, zeros_hbm, ones_hbm, out_hbm,
              idx0, idx1, ones_v, acc_sh, semi0, semi1):
    c = lax.axis_index("c")
    s = lax.axis_index("s")
    pltpu.sync_copy(zeros_hbm.at[pl.ds(s * SLC, SLC)],
                    acc_sh.at[pl.ds(s * SLC, SLC)])
    pltpu.sync_copy(ones_hbm, ones_v)
    plsc.subcore_barrier()
    base = (c * NS + s) * PER_W
    bufs = ((idx0, semi0), (idx1, semi1))

    def start_idx(i, buf):
        idx, semi = buf
        pltpu.async_copy(ei_hbm.at[1, pl.ds(base + i * B, B)], idx, semi)

    def wait_idx(i, buf):
        idx, semi = buf
        pltpu.make_async_copy(ei_hbm.at[1, pl.ds(base + i * B, B)],
                              idx, semi).wait()

    def chunk(i, b):
        wait_idx(i, bufs[b])
        pltpu.sync_copy(ones_v, acc_sh.at[bufs[b][0]], add=True)

        @pl.when(i + 2 < NSTEP)
        def _():
            start_idx(i + 2, bufs[b])

    start_idx(0, bufs[0])
    if NSTEP > 1:
        start_idx(1, bufs[1])

    def group(gi, carry):
        for b in range(2):
            chunk(gi * 2 + b, b)
        return carry

    lax.fori_loop(0, NSTEP // 2, group, 0)
    if NSTEP % 2:
        chunk(NSTEP - 1, (NSTEP - 1) % 2)
    plsc.subcore_barrier()
    pltpu.sync_copy(acc_sh.at[pl.ds(s * SLC, SLC)],
                    out_hbm.at[c, pl.ds(s * SLC, SLC)])


_sc_degree = pl.kernel(
    _deg_body,
    out_type=jax.ShapeDtypeStruct((NC, N16, H), jnp.float32),
    mesh=plsc.VectorSubcoreMesh(**_MESH),
    compiler_params=_SC_PARAMS,
    scratch_types=[
        pltpu.VMEM((B,), jnp.int32),
        pltpu.VMEM((B,), jnp.int32),
        pltpu.VMEM((B, H), jnp.float32),
        pltpu.VMEM_SHARED((N16, H), jnp.float32),
        pltpu.SemaphoreType.DMA,
        pltpu.SemaphoreType.DMA,
    ],
)


def _conv_body(ei_hbm, g_hbm, zeros_hbm, out_hbm,
               isrc0, idst0, rows0, isrc1, idst1, rows1, acc_sh,
               semg0, semg1, semi0, semi1):
    c = lax.axis_index("c")
    s = lax.axis_index("s")
    pltpu.sync_copy(zeros_hbm.at[pl.ds(s * SLC, SLC)],
                    acc_sh.at[pl.ds(s * SLC, SLC)])
    plsc.subcore_barrier()
    base = (c * NS + s) * PER_W
    bufs = ((isrc0, idst0, rows0, semg0, semi0),
            (isrc1, idst1, rows1, semg1, semi1))

    def start_idx(i, buf):
        isrc, idst, rows, semg, semi = buf
        off = base + i * B
        pltpu.async_copy(ei_hbm.at[0, pl.ds(off, B)], isrc, semi)
        pltpu.async_copy(ei_hbm.at[1, pl.ds(off, B)], idst, semi)

    def wait_idx(i, buf):
        isrc, idst, rows, semg, semi = buf
        off = base + i * B
        pltpu.make_async_copy(ei_hbm.at[0, pl.ds(off, B)], isrc, semi).wait()
        pltpu.make_async_copy(ei_hbm.at[1, pl.ds(off, B)], idst, semi).wait()

    def chunk(i, b):
        nxt = i + 1

        @pl.when(nxt < NSTEP)
        def _():
            wait_idx(nxt, bufs[1 - b])
            pltpu.async_copy(g_hbm.at[bufs[1 - b][0]], bufs[1 - b][2],
                             bufs[1 - b][3])

        isrc, idst, rows, semg, semi = bufs[b]
        pltpu.make_async_copy(g_hbm.at[isrc], rows, semg).wait()
        pltpu.sync_copy(rows, acc_sh.at[idst], add=True)

        @pl.when(i + 2 < NSTEP)
        def _():
            start_idx(i + 2, bufs[b])

    start_idx(0, bufs[0])
    wait_idx(0, bufs[0])
    pltpu.async_copy(g_hbm.at[bufs[0][0]], bufs[0][2], bufs[0][3])
    if NSTEP > 1:
        start_idx(1, bufs[1])

    def group(gi, carry):
        for b in range(2):
            chunk(gi * 2 + b, b)
        return carry

    lax.fori_loop(0, NSTEP // 2, group, 0)
    if NSTEP % 2:
        chunk(NSTEP - 1, (NSTEP - 1) % 2)
    plsc.subcore_barrier()
    pltpu.sync_copy(acc_sh.at[pl.ds(s * SLC, SLC)],
                    out_hbm.at[c, pl.ds(s * SLC, SLC)])


_sc_conv = pl.kernel(
    _conv_body,
    out_type=jax.ShapeDtypeStruct((NC, N16, H), jnp.float32),
    mesh=plsc.VectorSubcoreMesh(**_MESH),
    compiler_params=_SC_PARAMS,
    scratch_types=[
        pltpu.VMEM((B,), jnp.int32),
        pltpu.VMEM((B,), jnp.int32),
        pltpu.VMEM((B, H), jnp.float32),
        pltpu.VMEM((B,), jnp.int32),
        pltpu.VMEM((B,), jnp.int32),
        pltpu.VMEM((B, H), jnp.float32),
        pltpu.VMEM_SHARED((N16, H), jnp.float32),
        pltpu.SemaphoreType.DMA,
        pltpu.SemaphoreType.DMA,
        pltpu.SemaphoreType.DMA,
        pltpu.SemaphoreType.DMA,
    ],
)


def kernel(x, edge_index, W1, b1, W2, b2, Wc0, bc0, Wc1, bc1, Wp, bp):
    zeros16 = jnp.zeros((N16, H), jnp.float32)
    ones_b = jnp.ones((B, H), jnp.float32)

    b1r = b1.reshape(1, 256)
    b2r = b2.reshape(1, H)
    bc0r = bc0.reshape(1, H)
    bc1r = bc1.reshape(1, H)
    bpr = bp.reshape(1, 2)

    eye8 = jnp.eye(8, dtype=jnp.float32)
    Wc1bd = jnp.kron(eye8, Wc1)
    bc0t = jnp.tile(bc0, 8).reshape(1, 128)
    bc1t = jnp.tile(bc1, 8).reshape(1, 128)
    wp1 = jnp.sum(Wp, axis=1)
    wpbd = jnp.kron(eye8, wp1.reshape(H, 1))
    bps = jnp.full((1, 8), jnp.sum(bp), jnp.float32)

    degp = _sc_degree(edge_index, zeros16, ones_b)
    hw0 = _tc_mlp(x, W1, b1r, W2, b2r, Wc0)
    dinv, g0 = _tc_prep(degp.reshape(NC, M16, 128), hw0.reshape(M16, 128))
    acc0 = _sc_conv(edge_index, g0.reshape(N16, H), zeros16)
    g1 = _tc_epi(acc0.reshape(NC, M16, 128), g0, dinv, bc0t, Wc1bd)
    acc1 = _sc_conv(edge_index, g1.reshape(N16, H), zeros16)
    out2, hp = _tc_final(acc1.reshape(NC, M16, 128), g1, dinv, bc1t,
                         wpbd, bps)
    return (out2.reshape(N16)[:N], hp.reshape(N16, H)[:N])
